# baseline (speedup 1.0000x reference)
import functools

import jax
import jax.numpy as jnp
from jax.experimental import pallas as pl
from jax.experimental.pallas import tpu as pltpu

_INV_TEMP = 10.0
_EPS = 1e-12

_BM = 256


def _logits_kernel(x_ref, g_ref, o_ref, gn_ref):
    @pl.when(pl.program_id(0) == 0)
    def _():
        g = g_ref[...]
        gn = g / jnp.maximum(jnp.sqrt(jnp.sum(g * g, axis=1, keepdims=True)), _EPS)
        gn_ref[...] = gn.astype(jnp.bfloat16)

    x = x_ref[...]
    xs = x * (_INV_TEMP / jnp.maximum(jnp.sqrt(jnp.sum(x * x, axis=1, keepdims=True)), _EPS))
    o_ref[...] = jax.lax.dot_general(
        xs.astype(jnp.bfloat16),
        gn_ref[...],
        (((1,), (1,)), ((), ())),
        preferred_element_type=jnp.float32,
    )


@functools.partial(jax.jit, static_argnames=())
def kernel(x, group_features):
    m, k = x.shape
    n, _ = group_features.shape
    grid = (m // _BM,)
    return pl.pallas_call(
        _logits_kernel,
        grid=grid,
        in_specs=[
            pl.BlockSpec((_BM, k), lambda i: (i, 0)),
            pl.BlockSpec((n, k), lambda i: (0, 0)),
        ],
        out_specs=pl.BlockSpec((_BM, n), lambda i: (i, 0)),
        out_shape=jax.ShapeDtypeStruct((m, n), jnp.float32),
        scratch_shapes=[pltpu.VMEM((n, k), jnp.bfloat16)],
        compiler_params=pltpu.CompilerParams(
            dimension_semantics=("arbitrary",),
        ),
    )(x, group_features)
